# baseline (speedup 1.0000x reference)
import functools

import jax
import jax.numpy as jnp
from jax import lax
from jax.experimental import pallas as pl
from jax.experimental.pallas import tpu as pltpu
from jax.experimental.pallas import tpu_sc as plsc

NUM_FIELDS = 26
DIM = 32
BATCH = 16384
TOT = BATCH * NUM_FIELDS
NC, NS = 2, 16
NW = NC * NS
PER_W = TOT // NW
PERIOD = 208
CHUNK = 1664
NCHUNK = PER_W // CHUNK
GATHER_W = 128
NGATHER = CHUNK // GATHER_W


def _sc_gather(x_flat, table, off_tiled):
    mesh = plsc.VectorSubcoreMesh(core_axis_name="c", subcore_axis_name="s")

    @functools.partial(
        pl.kernel,
        mesh=mesh,
        out_type=jax.ShapeDtypeStruct((TOT, DIM), jnp.float32),
        compiler_params=pltpu.CompilerParams(use_tc_tiling_on_sc=False),
        scratch_types=[
            pltpu.VMEM((PERIOD,), jnp.int32),
            pltpu.VMEM((CHUNK,), jnp.int32),
            pltpu.VMEM((NGATHER, GATHER_W), jnp.int32),
            pltpu.VMEM((CHUNK, DIM), jnp.float32),
            pltpu.SemaphoreType.DMA,
        ],
    )
    def k(x_hbm, tab_hbm, off_hbm, out_hbm, off_v, xraw_v, idx_v, rows_v, sem):
        wid = lax.axis_index("s") * NC + lax.axis_index("c")
        base = wid * PER_W
        pltpu.sync_copy(off_hbm, off_v)

        def chunk_body(c, carry):
            start = base + c * CHUNK
            pltpu.sync_copy(x_hbm.at[pl.ds(start, CHUNK)], xraw_v)
            for r in range(NGATHER):
                for j in range(GATHER_W // 16):
                    q = r * GATHER_W + j * 16
                    idx_v[r, pl.ds(j * 16, 16)] = (
                        xraw_v[pl.ds(q, 16)] + off_v[pl.ds(q % PERIOD, 16)])
            handles = []
            for g in range(NGATHER):
                handles.append(pltpu.async_copy(
                    tab_hbm.at[idx_v.at[g]],
                    rows_v.at[pl.ds(g * GATHER_W, GATHER_W)],
                    sem))
            for h in handles:
                h.wait()
            pltpu.sync_copy(rows_v, out_hbm.at[pl.ds(start, CHUNK)])
            return carry

        lax.fori_loop(0, NCHUNK, chunk_body, 0)

    return k(x_flat, table, off_tiled)


def kernel(x, cat_emb_weight, categories_offset):
    x_flat = x.reshape(TOT)
    off_tiled = jnp.tile(categories_offset, PERIOD // NUM_FIELDS)
    out = _sc_gather(x_flat, cat_emb_weight, off_tiled)
    return out.reshape(BATCH, NUM_FIELDS, DIM)


# speedup vs baseline: 1.0027x; 1.0027x over previous
import functools

import jax
import jax.numpy as jnp
from jax import lax
from jax.experimental import pallas as pl
from jax.experimental.pallas import tpu as pltpu
from jax.experimental.pallas import tpu_sc as plsc

NUM_FIELDS = 26
DIM = 32
BATCH = 16384
TOT = BATCH * NUM_FIELDS
NC, NS = 2, 16
NW = NC * NS
PER_W = TOT // NW
PERIOD = 208
GATHER_W = 128
NROW = PER_W // GATHER_W
CHUNK_ROWS = 8
CHUNK = CHUNK_ROWS * GATHER_W
NCHUNK = PER_W // CHUNK
ADD_BLOCKS = 8
ROWS_PER_BLOCK = NROW // ADD_BLOCKS


def _sc_gather(x_flat, table, off_tiled):
    mesh = plsc.VectorSubcoreMesh(core_axis_name="c", subcore_axis_name="s")

    @functools.partial(
        pl.kernel,
        mesh=mesh,
        out_type=jax.ShapeDtypeStruct((TOT, DIM), jnp.float32),
        compiler_params=pltpu.CompilerParams(use_tc_tiling_on_sc=False),
        scratch_types=[
            pltpu.VMEM((PERIOD,), jnp.int32),
            pltpu.VMEM((PER_W,), jnp.int32),
            pltpu.VMEM((NROW, GATHER_W), jnp.int32),
            pltpu.VMEM((CHUNK, DIM), jnp.float32),
            pltpu.VMEM((CHUNK, DIM), jnp.float32),
            pltpu.SemaphoreType.DMA,
            pltpu.SemaphoreType.DMA,
            pltpu.SemaphoreType.DMA,
            pltpu.SemaphoreType.DMA,
        ],
    )
    def k(x_hbm, tab_hbm, off_hbm, out_hbm,
          off_v, xraw_v, idx_v, rows0_v, rows1_v,
          gsem0, gsem1, osem0, osem1):
        wid = lax.axis_index("s") * NC + lax.axis_index("c")
        base = wid * PER_W
        pltpu.sync_copy(off_hbm, off_v)
        pltpu.sync_copy(x_hbm.at[pl.ds(base, PER_W)], xraw_v)

        def add_block(i, carry):
            for r in range(ROWS_PER_BLOCK):
                row = idx_v.at[i * ROWS_PER_BLOCK + r]
                for j in range(GATHER_W // 16):
                    p = (r * GATHER_W + j * 16) % PERIOD
                    q = i * (ROWS_PER_BLOCK * GATHER_W) + r * GATHER_W + j * 16
                    row[pl.ds(j * 16, 16)] = (
                        xraw_v[pl.ds(q, 16)] + off_v[pl.ds(p, 16)])
            return carry

        lax.fori_loop(0, ADD_BLOCKS, add_block, 0)

        bufs = (rows0_v, rows1_v)
        gsems = (gsem0, gsem1)
        osems = (osem0, osem1)
        ghandles = [None, None]
        ohandles = [None, None]
        for c in range(NCHUNK):
            b = c % 2
            if ohandles[b] is not None:
                ohandles[b].wait()
            hs = []
            for g in range(CHUNK_ROWS):
                hs.append(pltpu.async_copy(
                    tab_hbm.at[idx_v.at[c * CHUNK_ROWS + g]],
                    bufs[b].at[pl.ds(g * GATHER_W, GATHER_W)],
                    gsems[b]))
            ghandles[b] = hs
            if c >= 1:
                pb = (c - 1) % 2
                for h in ghandles[pb]:
                    h.wait()
                ohandles[pb] = pltpu.async_copy(
                    bufs[pb],
                    out_hbm.at[pl.ds(base + (c - 1) * CHUNK, CHUNK)],
                    osems[pb])
        lb = (NCHUNK - 1) % 2
        for h in ghandles[lb]:
            h.wait()
        ohandles[lb] = pltpu.async_copy(
            bufs[lb],
            out_hbm.at[pl.ds(base + (NCHUNK - 1) * CHUNK, CHUNK)],
            osems[lb])
        ohandles[1 - lb].wait()
        ohandles[lb].wait()

    return k(x_flat, table, off_tiled)


def kernel(x, cat_emb_weight, categories_offset):
    x_flat = x.reshape(TOT)
    off_tiled = jnp.tile(categories_offset, PERIOD // NUM_FIELDS)
    out = _sc_gather(x_flat, cat_emb_weight, off_tiled)
    return out.reshape(BATCH, NUM_FIELDS, DIM)
